# baseline (speedup 1.0000x reference)
import functools

import jax
import jax.numpy as jnp
from jax import lax
from jax.experimental import pallas as pl
from jax.experimental.pallas import tpu as pltpu
from jax.experimental.pallas import tpu_sc as plsc

NC = 2
NS = 16
NW = NC * NS
K = 80
CW = 16
NP = 10240


def _sc_mesh():
    return plsc.VectorSubcoreMesh(core_axis_name="c", subcore_axis_name="s")


def _sc_degree(dst_grp, ones_rows, zeros_cnt):
    nchunks = dst_grp.shape[1]
    rpt = NP // NS

    @functools.partial(
        pl.kernel,
        mesh=_sc_mesh(),
        out_type=jax.ShapeDtypeStruct((NC, NP, CW), jnp.float32),
        scratch_types=[
            pltpu.VMEM((2, 1, K), jnp.int32),
            pltpu.VMEM((K, CW), jnp.float32),
            pltpu.VMEM_SHARED((NP, CW), jnp.float32),
            pltpu.SemaphoreType.DMA,
        ],
    )
    def kern(dst_hbm, ones_hbm, zeros_hbm, out_hbm, dst_v, ones_v, cnt_sh,
             sem):
        cid = lax.axis_index("c")
        sid = lax.axis_index("s")
        wid = cid * NS + sid
        pltpu.sync_copy(ones_hbm, ones_v)
        pltpu.sync_copy(zeros_hbm, cnt_sh.at[pl.ds(sid * rpt, rpt)])
        plsc.subcore_barrier()

        pltpu.async_copy(dst_hbm.at[wid, pl.ds(0, 1)], dst_v.at[0], sem)

        def body(i, carry):
            j = 2 * i
            pltpu.async_copy(dst_hbm.at[wid, pl.ds(j + 1, 1)], dst_v.at[1],
                             sem)
            pltpu.make_async_copy(dst_hbm.at[wid, pl.ds(j, 1)], dst_v.at[0],
                                  sem).wait()
            pltpu.sync_copy(ones_v, cnt_sh.at[dst_v.at[0, 0]], add=True)
            pltpu.async_copy(dst_hbm.at[wid, pl.ds(j + 2, 1)], dst_v.at[0],
                             sem)
            pltpu.make_async_copy(dst_hbm.at[wid, pl.ds(j + 1, 1)],
                                  dst_v.at[1], sem).wait()
            pltpu.sync_copy(ones_v, cnt_sh.at[dst_v.at[1, 0]], add=True)
            return carry

        lax.fori_loop(0, (nchunks - 1) // 2, body, 0)
        last = nchunks - 1
        pltpu.make_async_copy(dst_hbm.at[wid, pl.ds(last, 1)], dst_v.at[0],
                              sem).wait()
        pltpu.sync_copy(ones_v, cnt_sh.at[dst_v.at[0, 0]], add=True)
        plsc.subcore_barrier()
        pltpu.sync_copy(cnt_sh.at[pl.ds(sid * rpt, rpt)],
                        out_hbm.at[cid, pl.ds(sid * rpt, rpt)])

    return kern(dst_grp, ones_rows, zeros_cnt)


def _sc_edge_agg(y, src_grp, dst_grp, zeros_rows):
    n, d = y.shape
    nchunks = src_grp.shape[1]
    rpt = NP // NS

    @functools.partial(
        pl.kernel,
        mesh=_sc_mesh(),
        out_type=jax.ShapeDtypeStruct((NC, NP, d), jnp.float32),
        scratch_types=[
            pltpu.VMEM((nchunks, K), jnp.int32),
            pltpu.VMEM((nchunks, K), jnp.int32),
            pltpu.VMEM((K // 2, d), jnp.float32),
            pltpu.VMEM((K // 2, d), jnp.float32),
            pltpu.VMEM_SHARED((NP, d), jnp.float32),
            pltpu.SemaphoreType.DMA,
            pltpu.SemaphoreType.DMA,
        ],
    )
    def kern(y_hbm, src_hbm, dst_hbm, zeros_hbm, out_hbm,
             src_v, dst_v, rows_a, rows_b, agg_sh, sema, semb):
        cid = lax.axis_index("c")
        sid = lax.axis_index("s")
        wid = cid * NS + sid
        pltpu.sync_copy(src_hbm.at[wid], src_v)
        pltpu.sync_copy(dst_hbm.at[wid], dst_v)
        pltpu.sync_copy(zeros_hbm, agg_sh.at[pl.ds(sid * rpt, rpt)])
        plsc.subcore_barrier()

        kh = K // 2

        def body(j, carry):
            cpa = pltpu.async_copy(y_hbm.at[src_v.at[j, pl.ds(0, kh)]],
                                   rows_a, sema)
            cpa.wait()
            cpb = pltpu.async_copy(y_hbm.at[src_v.at[j, pl.ds(kh, kh)]],
                                   rows_b, semb)
            pltpu.sync_copy(rows_a, agg_sh.at[dst_v.at[j, pl.ds(0, kh)]],
                            add=True)
            cpb.wait()
            pltpu.sync_copy(rows_b, agg_sh.at[dst_v.at[j, pl.ds(kh, kh)]],
                            add=True)
            return carry

        lax.fori_loop(0, nchunks, body, 0)
        plsc.subcore_barrier()
        pltpu.sync_copy(agg_sh.at[pl.ds(sid * rpt, rpt)],
                        out_hbm.at[cid, pl.ds(sid * rpt, rpt)])

    return kern(y, src_grp, dst_grp, zeros_rows)


def _tc_matmul(x, W):
    n, f = x.shape
    h = W.shape[1]
    R = 1000

    def body(x_ref, w_ref, o_ref):
        o_ref[...] = jnp.dot(x_ref[...], w_ref[...],
                             preferred_element_type=jnp.float32)

    return pl.pallas_call(
        body,
        grid=(n // R,),
        in_specs=[pl.BlockSpec((R, f), lambda i: (i, 0)),
                  pl.BlockSpec((f, h), lambda i: (0, 0))],
        out_specs=pl.BlockSpec((R, h), lambda i: (i, 0)),
        out_shape=jax.ShapeDtypeStruct((n, h), jnp.float32),
    )(x, W)


def _tc_prep(degp, xw1):
    n, h = xw1.shape
    R = 1000

    def body(deg_ref, xw_ref, y_ref, dinv_ref):
        dtot = deg_ref[0, :, 0:1] + deg_ref[1, :, 0:1] + 1.0
        dv = lax.rsqrt(dtot)
        dinv_ref[...] = dv
        y_ref[...] = xw_ref[...] * dv

    return pl.pallas_call(
        body,
        grid=(n // R,),
        in_specs=[pl.BlockSpec((NC, R, CW), lambda i: (0, i, 0)),
                  pl.BlockSpec((R, h), lambda i: (i, 0))],
        out_specs=[pl.BlockSpec((R, h), lambda i: (i, 0)),
                   pl.BlockSpec((R, 1), lambda i: (i, 0))],
        out_shape=[jax.ShapeDtypeStruct((n, h), jnp.float32),
                   jax.ShapeDtypeStruct((n, 1), jnp.float32)],
    )(degp, xw1)


def _tc_mid(s1, y1, dinv, b1, W2):
    n, h = y1.shape
    R = 1000

    def body(s_ref, y_ref, dv_ref, b_ref, w_ref, h1_ref, y2_ref):
        dv = dv_ref[...]
        agg = dv * (s_ref[0] + s_ref[1] + y_ref[...]) + b_ref[...]
        h1 = jnp.maximum(agg, 0.0)
        h1_ref[...] = h1
        y2_ref[...] = jnp.dot(h1, w_ref[...],
                              preferred_element_type=jnp.float32) * dv

    return pl.pallas_call(
        body,
        grid=(n // R,),
        in_specs=[pl.BlockSpec((NC, R, h), lambda i: (0, i, 0)),
                  pl.BlockSpec((R, h), lambda i: (i, 0)),
                  pl.BlockSpec((R, 1), lambda i: (i, 0)),
                  pl.BlockSpec((1, h), lambda i: (0, 0)),
                  pl.BlockSpec((h, h), lambda i: (0, 0))],
        out_specs=[pl.BlockSpec((R, h), lambda i: (i, 0)),
                   pl.BlockSpec((R, h), lambda i: (i, 0))],
        out_shape=[jax.ShapeDtypeStruct((n, h), jnp.float32),
                   jax.ShapeDtypeStruct((n, h), jnp.float32)],
    )(s1, y1, dinv, b1, W2)


def _tc_final(s2, y2, h1, dinv, b2, gamma, beta, hprev, W_ih, W_hh, bih, bhh):
    n, h = h1.shape
    h3 = W_ih.shape[0]
    R = 1000

    def body(s_ref, y_ref, h1_ref, dv_ref, b2_ref, g_ref, be_ref, hp_ref,
             wih_ref, whh_ref, bih_ref, bhh_ref, out_ref):
        dv = dv_ref[...]
        agg = dv * (s_ref[0] + s_ref[1] + y_ref[...]) + b2_ref[...]
        h2 = h1_ref[...] + jnp.maximum(agg, 0.0)
        mu = jnp.mean(h2, axis=-1, keepdims=True)
        var = jnp.mean((h2 - mu) ** 2, axis=-1, keepdims=True)
        hln = (h2 - mu) * lax.rsqrt(var + 1e-5) * g_ref[...] + be_ref[...]
        hprev_b = hp_ref[...]
        gi = lax.dot_general(hln, wih_ref[...], (((1,), (1,)), ((), ())),
                             preferred_element_type=jnp.float32) + bih_ref[...]
        gh = lax.dot_general(hprev_b, whh_ref[...], (((1,), (1,)), ((), ())),
                             preferred_element_type=jnp.float32) + bhh_ref[...]
        r = jax.nn.sigmoid(gi[:, :h] + gh[:, :h])
        z = jax.nn.sigmoid(gi[:, h:2 * h] + gh[:, h:2 * h])
        nn_ = jnp.tanh(gi[:, 2 * h:] + r * gh[:, 2 * h:])
        out_ref[...] = (1.0 - z) * nn_ + z * hprev_b

    return pl.pallas_call(
        body,
        grid=(n // R,),
        in_specs=[pl.BlockSpec((NC, R, h), lambda i: (0, i, 0)),
                  pl.BlockSpec((R, h), lambda i: (i, 0)),
                  pl.BlockSpec((R, h), lambda i: (i, 0)),
                  pl.BlockSpec((R, 1), lambda i: (i, 0)),
                  pl.BlockSpec((1, h), lambda i: (0, 0)),
                  pl.BlockSpec((1, h), lambda i: (0, 0)),
                  pl.BlockSpec((1, h), lambda i: (0, 0)),
                  pl.BlockSpec((R, h), lambda i: (i, 0)),
                  pl.BlockSpec((h3, h), lambda i: (0, 0)),
                  pl.BlockSpec((h3, h), lambda i: (0, 0)),
                  pl.BlockSpec((1, h3), lambda i: (0, 0)),
                  pl.BlockSpec((1, h3), lambda i: (0, 0))],
        out_specs=pl.BlockSpec((R, h), lambda i: (i, 0)),
        out_shape=jax.ShapeDtypeStruct((n, h), jnp.float32),
    )(s2, y2, h1, dinv, b2, gamma, beta, hprev, W_ih, W_hh, bih, bhh)


def kernel(x, edge_index, hidden_state, W1, b1, W2, b2,
           ln_gamma, ln_beta, W_ih, W_hh, b_ih, b_hh):
    n = x.shape[0]
    hdim = W1.shape[1]
    e = edge_index.shape[1]
    nchunks = e // (NW * K)

    src_grp = edge_index[0].reshape(NW, nchunks, K)
    dst_grp = edge_index[1].reshape(NW, nchunks, K)
    dst_deg = dst_grp

    ones_rows = jnp.concatenate(
        [jnp.ones((K, 1), jnp.float32), jnp.zeros((K, CW - 1), jnp.float32)],
        axis=1)
    zeros_cnt = jnp.zeros((NP // NS, CW), jnp.float32)
    zeros_rows = jnp.zeros((NP // NS, hdim), jnp.float32)

    degp = _sc_degree(dst_deg, ones_rows, zeros_cnt)
    xw1 = _tc_matmul(x, W1)
    y1, dinv = _tc_prep(degp, xw1)
    s1 = _sc_edge_agg(y1, src_grp, dst_grp, zeros_rows)
    h1, y2 = _tc_mid(s1, y1, dinv, b1.reshape(1, hdim), W2)
    s2 = _sc_edge_agg(y2, src_grp, dst_grp, zeros_rows)
    hnew = _tc_final(s2, y2, h1, dinv, b2.reshape(1, hdim),
                     ln_gamma.reshape(1, hdim), ln_beta.reshape(1, hdim),
                     hidden_state[0], W_ih, W_hh,
                     b_ih.reshape(1, -1), b_hh.reshape(1, -1))
    return hnew, hnew[None, :, :]


# speedup vs baseline: 1.1892x; 1.1892x over previous
import functools

import jax
import jax.numpy as jnp
from jax import lax
from jax.experimental import pallas as pl
from jax.experimental.pallas import tpu as pltpu
from jax.experimental.pallas import tpu_sc as plsc

NC = 2
NS = 16
NW = NC * NS
K = 80
CW = 16
NP = 10240


def _sc_mesh():
    return plsc.VectorSubcoreMesh(core_axis_name="c", subcore_axis_name="s")


def _sc_degree(dst_grp, ones_rows, zeros_cnt):
    nchunks = dst_grp.shape[1]
    rpt = NP // NS

    @functools.partial(
        pl.kernel,
        mesh=_sc_mesh(),
        out_type=jax.ShapeDtypeStruct((NC, NP, CW), jnp.float32),
        scratch_types=[
            pltpu.VMEM((2, 1, K), jnp.int32),
            pltpu.VMEM((K, CW), jnp.float32),
            pltpu.VMEM_SHARED((NP, CW), jnp.float32),
            pltpu.SemaphoreType.DMA,
        ],
    )
    def kern(dst_hbm, ones_hbm, zeros_hbm, out_hbm, dst_v, ones_v, cnt_sh,
             sem):
        cid = lax.axis_index("c")
        sid = lax.axis_index("s")
        wid = cid * NS + sid
        pltpu.sync_copy(ones_hbm, ones_v)
        pltpu.sync_copy(zeros_hbm, cnt_sh.at[pl.ds(sid * rpt, rpt)])
        plsc.subcore_barrier()

        pltpu.async_copy(dst_hbm.at[wid, pl.ds(0, 1)], dst_v.at[0], sem)

        def body(i, carry):
            j = 2 * i
            pltpu.async_copy(dst_hbm.at[wid, pl.ds(j + 1, 1)], dst_v.at[1],
                             sem)
            pltpu.make_async_copy(dst_hbm.at[wid, pl.ds(j, 1)], dst_v.at[0],
                                  sem).wait()
            pltpu.sync_copy(ones_v, cnt_sh.at[dst_v.at[0, 0]], add=True)
            pltpu.async_copy(dst_hbm.at[wid, pl.ds(j + 2, 1)], dst_v.at[0],
                             sem)
            pltpu.make_async_copy(dst_hbm.at[wid, pl.ds(j + 1, 1)],
                                  dst_v.at[1], sem).wait()
            pltpu.sync_copy(ones_v, cnt_sh.at[dst_v.at[1, 0]], add=True)
            return carry

        lax.fori_loop(0, (nchunks - 1) // 2, body, 0)
        last = nchunks - 1
        pltpu.make_async_copy(dst_hbm.at[wid, pl.ds(last, 1)], dst_v.at[0],
                              sem).wait()
        pltpu.sync_copy(ones_v, cnt_sh.at[dst_v.at[0, 0]], add=True)
        plsc.subcore_barrier()
        pltpu.sync_copy(cnt_sh.at[pl.ds(sid * rpt, rpt)],
                        out_hbm.at[cid, pl.ds(sid * rpt, rpt)])

    return kern(dst_grp, ones_rows, zeros_cnt)


def _sc_edge_agg(y, src_grp, dst_grp, zeros_rows):
    n, d = y.shape
    nchunks = src_grp.shape[1]
    rpt = NP // NS

    @functools.partial(
        pl.kernel,
        mesh=_sc_mesh(),
        out_type=jax.ShapeDtypeStruct((NC, NP, d), jnp.float32),
        scratch_types=[
            pltpu.VMEM((nchunks, K), jnp.int32),
            pltpu.VMEM((nchunks, K), jnp.int32),
            pltpu.VMEM((K, d), jnp.float32),
            pltpu.VMEM_SHARED((NP, d), jnp.float32),
            pltpu.SemaphoreType.DMA,
        ],
    )
    def kern(y_hbm, src_hbm, dst_hbm, zeros_hbm, out_hbm,
             src_v, dst_v, rows_v, agg_sh, sem):
        cid = lax.axis_index("c")
        sid = lax.axis_index("s")
        wid = cid * NS + sid
        pltpu.sync_copy(src_hbm.at[wid], src_v)
        pltpu.sync_copy(dst_hbm.at[wid], dst_v)
        pltpu.sync_copy(zeros_hbm, agg_sh.at[pl.ds(sid * rpt, rpt)])
        plsc.subcore_barrier()

        def body(j, carry):
            pltpu.async_copy(y_hbm.at[src_v.at[j]], rows_v, sem).wait()
            pltpu.sync_copy(rows_v, agg_sh.at[dst_v.at[j]], add=True)
            return carry

        lax.fori_loop(0, nchunks, body, 0)
        plsc.subcore_barrier()
        pltpu.sync_copy(agg_sh.at[pl.ds(sid * rpt, rpt)],
                        out_hbm.at[cid, pl.ds(sid * rpt, rpt)])

    return kern(y, src_grp, dst_grp, zeros_rows)


def _tc_matmul(x, W):
    n, f = x.shape
    h = W.shape[1]
    R = 1000

    def body(x_ref, w_ref, o_ref):
        o_ref[...] = jnp.dot(x_ref[...], w_ref[...],
                             preferred_element_type=jnp.float32)

    return pl.pallas_call(
        body,
        grid=(n // R,),
        in_specs=[pl.BlockSpec((R, f), lambda i: (i, 0)),
                  pl.BlockSpec((f, h), lambda i: (0, 0))],
        out_specs=pl.BlockSpec((R, h), lambda i: (i, 0)),
        out_shape=jax.ShapeDtypeStruct((n, h), jnp.float32),
    )(x, W)


def _tc_prep(degp, xw1):
    n, h = xw1.shape
    R = 1000

    def body(deg_ref, xw_ref, y_ref, dinv_ref):
        dtot = deg_ref[0, :, 0:1] + deg_ref[1, :, 0:1] + 1.0
        dv = lax.rsqrt(dtot)
        dinv_ref[...] = dv
        y_ref[...] = xw_ref[...] * dv

    return pl.pallas_call(
        body,
        grid=(n // R,),
        in_specs=[pl.BlockSpec((NC, R, CW), lambda i: (0, i, 0)),
                  pl.BlockSpec((R, h), lambda i: (i, 0))],
        out_specs=[pl.BlockSpec((R, h), lambda i: (i, 0)),
                   pl.BlockSpec((R, 1), lambda i: (i, 0))],
        out_shape=[jax.ShapeDtypeStruct((n, h), jnp.float32),
                   jax.ShapeDtypeStruct((n, 1), jnp.float32)],
    )(degp, xw1)


def _tc_mid(s1, y1, dinv, b1, W2):
    n, h = y1.shape
    R = 1000

    def body(s_ref, y_ref, dv_ref, b_ref, w_ref, h1_ref, y2_ref):
        dv = dv_ref[...]
        agg = dv * (s_ref[0] + s_ref[1] + y_ref[...]) + b_ref[...]
        h1 = jnp.maximum(agg, 0.0)
        h1_ref[...] = h1
        y2_ref[...] = jnp.dot(h1, w_ref[...],
                              preferred_element_type=jnp.float32) * dv

    return pl.pallas_call(
        body,
        grid=(n // R,),
        in_specs=[pl.BlockSpec((NC, R, h), lambda i: (0, i, 0)),
                  pl.BlockSpec((R, h), lambda i: (i, 0)),
                  pl.BlockSpec((R, 1), lambda i: (i, 0)),
                  pl.BlockSpec((1, h), lambda i: (0, 0)),
                  pl.BlockSpec((h, h), lambda i: (0, 0))],
        out_specs=[pl.BlockSpec((R, h), lambda i: (i, 0)),
                   pl.BlockSpec((R, h), lambda i: (i, 0))],
        out_shape=[jax.ShapeDtypeStruct((n, h), jnp.float32),
                   jax.ShapeDtypeStruct((n, h), jnp.float32)],
    )(s1, y1, dinv, b1, W2)


def _tc_final(s2, y2, h1, dinv, b2, gamma, beta, hprev, W_ih, W_hh, bih, bhh):
    n, h = h1.shape
    h3 = W_ih.shape[0]
    R = 1000

    def body(s_ref, y_ref, h1_ref, dv_ref, b2_ref, g_ref, be_ref, hp_ref,
             wih_ref, whh_ref, bih_ref, bhh_ref, out_ref):
        dv = dv_ref[...]
        agg = dv * (s_ref[0] + s_ref[1] + y_ref[...]) + b2_ref[...]
        h2 = h1_ref[...] + jnp.maximum(agg, 0.0)
        mu = jnp.mean(h2, axis=-1, keepdims=True)
        var = jnp.mean((h2 - mu) ** 2, axis=-1, keepdims=True)
        hln = (h2 - mu) * lax.rsqrt(var + 1e-5) * g_ref[...] + be_ref[...]
        hprev_b = hp_ref[...]
        gi = lax.dot_general(hln, wih_ref[...], (((1,), (1,)), ((), ())),
                             preferred_element_type=jnp.float32) + bih_ref[...]
        gh = lax.dot_general(hprev_b, whh_ref[...], (((1,), (1,)), ((), ())),
                             preferred_element_type=jnp.float32) + bhh_ref[...]
        r = jax.nn.sigmoid(gi[:, :h] + gh[:, :h])
        z = jax.nn.sigmoid(gi[:, h:2 * h] + gh[:, h:2 * h])
        nn_ = jnp.tanh(gi[:, 2 * h:] + r * gh[:, 2 * h:])
        out_ref[...] = (1.0 - z) * nn_ + z * hprev_b

    return pl.pallas_call(
        body,
        grid=(n // R,),
        in_specs=[pl.BlockSpec((NC, R, h), lambda i: (0, i, 0)),
                  pl.BlockSpec((R, h), lambda i: (i, 0)),
                  pl.BlockSpec((R, h), lambda i: (i, 0)),
                  pl.BlockSpec((R, 1), lambda i: (i, 0)),
                  pl.BlockSpec((1, h), lambda i: (0, 0)),
                  pl.BlockSpec((1, h), lambda i: (0, 0)),
                  pl.BlockSpec((1, h), lambda i: (0, 0)),
                  pl.BlockSpec((R, h), lambda i: (i, 0)),
                  pl.BlockSpec((h3, h), lambda i: (0, 0)),
                  pl.BlockSpec((h3, h), lambda i: (0, 0)),
                  pl.BlockSpec((1, h3), lambda i: (0, 0)),
                  pl.BlockSpec((1, h3), lambda i: (0, 0))],
        out_specs=pl.BlockSpec((R, h), lambda i: (i, 0)),
        out_shape=jax.ShapeDtypeStruct((n, h), jnp.float32),
    )(s2, y2, h1, dinv, b2, gamma, beta, hprev, W_ih, W_hh, bih, bhh)


def kernel(x, edge_index, hidden_state, W1, b1, W2, b2,
           ln_gamma, ln_beta, W_ih, W_hh, b_ih, b_hh):
    n = x.shape[0]
    hdim = W1.shape[1]
    e = edge_index.shape[1]
    nchunks = e // (NW * K)

    src_grp = edge_index[0].reshape(NW, nchunks, K)
    dst_grp = edge_index[1].reshape(NW, nchunks, K)
    dst_deg = dst_grp

    ones_rows = jnp.concatenate(
        [jnp.ones((K, 1), jnp.float32), jnp.zeros((K, CW - 1), jnp.float32)],
        axis=1)
    zeros_cnt = jnp.zeros((NP // NS, CW), jnp.float32)
    zeros_rows = jnp.zeros((NP // NS, hdim), jnp.float32)

    degp = _sc_degree(dst_deg, ones_rows, zeros_cnt)
    xw1 = _tc_matmul(x, W1)
    y1, dinv = _tc_prep(degp, xw1)
    s1 = _sc_edge_agg(y1, src_grp, dst_grp, zeros_rows)
    h1, y2 = _tc_mid(s1, y1, dinv, b1.reshape(1, hdim), W2)
    s2 = _sc_edge_agg(y2, src_grp, dst_grp, zeros_rows)
    hnew = _tc_final(s2, y2, h1, dinv, b2.reshape(1, hdim),
                     ln_gamma.reshape(1, hdim), ln_beta.reshape(1, hdim),
                     hidden_state[0], W_ih, W_hh,
                     b_ih.reshape(1, -1), b_hh.reshape(1, -1))
    return hnew, hnew[None, :, :]


# speedup vs baseline: 1.2439x; 1.0460x over previous
import functools

import jax
import jax.numpy as jnp
from jax import lax
from jax.experimental import pallas as pl
from jax.experimental.pallas import tpu as pltpu
from jax.experimental.pallas import tpu_sc as plsc

NC = 2
NS = 16
NW = NC * NS
K = 80
CW = 16
NP = 10240


def _sc_mesh():
    return plsc.VectorSubcoreMesh(core_axis_name="c", subcore_axis_name="s")


def _sc_degree(dst_grp, ones_rows, zeros_cnt):
    nchunks = dst_grp.shape[1]
    rpt = NP // NS

    @functools.partial(
        pl.kernel,
        mesh=_sc_mesh(),
        out_type=jax.ShapeDtypeStruct((NC, NP, CW), jnp.float32),
        scratch_types=[
            pltpu.VMEM((nchunks, K), jnp.int32),
            pltpu.VMEM((K, CW), jnp.float32),
            pltpu.VMEM_SHARED((NP, CW), jnp.float32),
        ],
    )
    def kern(dst_hbm, ones_hbm, zeros_hbm, out_hbm, dst_v, ones_v, cnt_sh):
        cid = lax.axis_index("c")
        sid = lax.axis_index("s")
        wid = cid * NS + sid
        pltpu.sync_copy(dst_hbm.at[wid], dst_v)
        pltpu.sync_copy(ones_hbm, ones_v)
        pltpu.sync_copy(zeros_hbm, cnt_sh.at[pl.ds(sid * rpt, rpt)])
        plsc.subcore_barrier()

        def body(j, carry):
            pltpu.sync_copy(ones_v, cnt_sh.at[dst_v.at[j]], add=True)
            return carry

        lax.fori_loop(0, nchunks, body, 0)
        plsc.subcore_barrier()
        pltpu.sync_copy(cnt_sh.at[pl.ds(sid * rpt, rpt)],
                        out_hbm.at[cid, pl.ds(sid * rpt, rpt)])

    return kern(dst_grp, ones_rows, zeros_cnt)


def _sc_edge_agg(y, src_grp, dst_grp, zeros_rows):
    n, d = y.shape
    nchunks = src_grp.shape[1]
    rpt = NP // NS

    @functools.partial(
        pl.kernel,
        mesh=_sc_mesh(),
        out_type=jax.ShapeDtypeStruct((NC, NP, d), jnp.float32),
        scratch_types=[
            pltpu.VMEM((nchunks, K), jnp.int32),
            pltpu.VMEM((nchunks, K), jnp.int32),
            pltpu.VMEM((K, d), jnp.float32),
            pltpu.VMEM_SHARED((NP, d), jnp.float32),
            pltpu.SemaphoreType.DMA,
        ],
    )
    def kern(y_hbm, src_hbm, dst_hbm, zeros_hbm, out_hbm,
             src_v, dst_v, rows_v, agg_sh, sem):
        cid = lax.axis_index("c")
        sid = lax.axis_index("s")
        wid = cid * NS + sid
        pltpu.sync_copy(src_hbm.at[wid], src_v)
        pltpu.sync_copy(dst_hbm.at[wid], dst_v)
        pltpu.sync_copy(zeros_hbm, agg_sh.at[pl.ds(sid * rpt, rpt)])
        plsc.subcore_barrier()

        def body(j, carry):
            pltpu.async_copy(y_hbm.at[src_v.at[j]], rows_v, sem).wait()
            pltpu.sync_copy(rows_v, agg_sh.at[dst_v.at[j]], add=True)
            return carry

        lax.fori_loop(0, nchunks, body, 0)
        plsc.subcore_barrier()
        pltpu.sync_copy(agg_sh.at[pl.ds(sid * rpt, rpt)],
                        out_hbm.at[cid, pl.ds(sid * rpt, rpt)])

    return kern(y, src_grp, dst_grp, zeros_rows)


def _tc_matmul(x, W):
    n, f = x.shape
    h = W.shape[1]
    R = 1000

    def body(x_ref, w_ref, o_ref):
        o_ref[...] = jnp.dot(x_ref[...], w_ref[...],
                             preferred_element_type=jnp.float32)

    return pl.pallas_call(
        body,
        grid=(n // R,),
        in_specs=[pl.BlockSpec((R, f), lambda i: (i, 0)),
                  pl.BlockSpec((f, h), lambda i: (0, 0))],
        out_specs=pl.BlockSpec((R, h), lambda i: (i, 0)),
        out_shape=jax.ShapeDtypeStruct((n, h), jnp.float32),
    )(x, W)


def _tc_prep(degp, xw1):
    n, h = xw1.shape
    R = 1000

    def body(deg_ref, xw_ref, y_ref, dinv_ref):
        dtot = deg_ref[0, :, 0:1] + deg_ref[1, :, 0:1] + 1.0
        dv = lax.rsqrt(dtot)
        dinv_ref[...] = dv
        y_ref[...] = xw_ref[...] * dv

    return pl.pallas_call(
        body,
        grid=(n // R,),
        in_specs=[pl.BlockSpec((NC, R, CW), lambda i: (0, i, 0)),
                  pl.BlockSpec((R, h), lambda i: (i, 0))],
        out_specs=[pl.BlockSpec((R, h), lambda i: (i, 0)),
                   pl.BlockSpec((R, 1), lambda i: (i, 0))],
        out_shape=[jax.ShapeDtypeStruct((n, h), jnp.float32),
                   jax.ShapeDtypeStruct((n, 1), jnp.float32)],
    )(degp, xw1)


def _tc_mid(s1, y1, dinv, b1, W2):
    n, h = y1.shape
    R = 1000

    def body(s_ref, y_ref, dv_ref, b_ref, w_ref, h1_ref, y2_ref):
        dv = dv_ref[...]
        agg = dv * (s_ref[0] + s_ref[1] + y_ref[...]) + b_ref[...]
        h1 = jnp.maximum(agg, 0.0)
        h1_ref[...] = h1
        y2_ref[...] = jnp.dot(h1, w_ref[...],
                              preferred_element_type=jnp.float32) * dv

    return pl.pallas_call(
        body,
        grid=(n // R,),
        in_specs=[pl.BlockSpec((NC, R, h), lambda i: (0, i, 0)),
                  pl.BlockSpec((R, h), lambda i: (i, 0)),
                  pl.BlockSpec((R, 1), lambda i: (i, 0)),
                  pl.BlockSpec((1, h), lambda i: (0, 0)),
                  pl.BlockSpec((h, h), lambda i: (0, 0))],
        out_specs=[pl.BlockSpec((R, h), lambda i: (i, 0)),
                   pl.BlockSpec((R, h), lambda i: (i, 0))],
        out_shape=[jax.ShapeDtypeStruct((n, h), jnp.float32),
                   jax.ShapeDtypeStruct((n, h), jnp.float32)],
    )(s1, y1, dinv, b1, W2)


def _tc_final(s2, y2, h1, dinv, b2, gamma, beta, hprev, W_ih, W_hh, bih, bhh):
    n, h = h1.shape
    h3 = W_ih.shape[0]
    R = 1000

    def body(s_ref, y_ref, h1_ref, dv_ref, b2_ref, g_ref, be_ref, hp_ref,
             wih_ref, whh_ref, bih_ref, bhh_ref, out_ref):
        dv = dv_ref[...]
        agg = dv * (s_ref[0] + s_ref[1] + y_ref[...]) + b2_ref[...]
        h2 = h1_ref[...] + jnp.maximum(agg, 0.0)
        mu = jnp.mean(h2, axis=-1, keepdims=True)
        var = jnp.mean((h2 - mu) ** 2, axis=-1, keepdims=True)
        hln = (h2 - mu) * lax.rsqrt(var + 1e-5) * g_ref[...] + be_ref[...]
        hprev_b = hp_ref[...]
        gi = lax.dot_general(hln, wih_ref[...], (((1,), (1,)), ((), ())),
                             preferred_element_type=jnp.float32) + bih_ref[...]
        gh = lax.dot_general(hprev_b, whh_ref[...], (((1,), (1,)), ((), ())),
                             preferred_element_type=jnp.float32) + bhh_ref[...]
        r = jax.nn.sigmoid(gi[:, :h] + gh[:, :h])
        z = jax.nn.sigmoid(gi[:, h:2 * h] + gh[:, h:2 * h])
        nn_ = jnp.tanh(gi[:, 2 * h:] + r * gh[:, 2 * h:])
        out_ref[...] = (1.0 - z) * nn_ + z * hprev_b

    return pl.pallas_call(
        body,
        grid=(n // R,),
        in_specs=[pl.BlockSpec((NC, R, h), lambda i: (0, i, 0)),
                  pl.BlockSpec((R, h), lambda i: (i, 0)),
                  pl.BlockSpec((R, h), lambda i: (i, 0)),
                  pl.BlockSpec((R, 1), lambda i: (i, 0)),
                  pl.BlockSpec((1, h), lambda i: (0, 0)),
                  pl.BlockSpec((1, h), lambda i: (0, 0)),
                  pl.BlockSpec((1, h), lambda i: (0, 0)),
                  pl.BlockSpec((R, h), lambda i: (i, 0)),
                  pl.BlockSpec((h3, h), lambda i: (0, 0)),
                  pl.BlockSpec((h3, h), lambda i: (0, 0)),
                  pl.BlockSpec((1, h3), lambda i: (0, 0)),
                  pl.BlockSpec((1, h3), lambda i: (0, 0))],
        out_specs=pl.BlockSpec((R, h), lambda i: (i, 0)),
        out_shape=jax.ShapeDtypeStruct((n, h), jnp.float32),
    )(s2, y2, h1, dinv, b2, gamma, beta, hprev, W_ih, W_hh, bih, bhh)


def kernel(x, edge_index, hidden_state, W1, b1, W2, b2,
           ln_gamma, ln_beta, W_ih, W_hh, b_ih, b_hh):
    n = x.shape[0]
    hdim = W1.shape[1]
    e = edge_index.shape[1]
    nchunks = e // (NW * K)

    src_grp = edge_index[0].reshape(NW, nchunks, K)
    dst_grp = edge_index[1].reshape(NW, nchunks, K)
    dst_deg = dst_grp

    ones_rows = jnp.concatenate(
        [jnp.ones((K, 1), jnp.float32), jnp.zeros((K, CW - 1), jnp.float32)],
        axis=1)
    zeros_cnt = jnp.zeros((NP // NS, CW), jnp.float32)
    zeros_rows = jnp.zeros((NP // NS, hdim), jnp.float32)

    degp = _sc_degree(dst_deg, ones_rows, zeros_cnt)
    xw1 = _tc_matmul(x, W1)
    y1, dinv = _tc_prep(degp, xw1)
    s1 = _sc_edge_agg(y1, src_grp, dst_grp, zeros_rows)
    h1, y2 = _tc_mid(s1, y1, dinv, b1.reshape(1, hdim), W2)
    s2 = _sc_edge_agg(y2, src_grp, dst_grp, zeros_rows)
    hnew = _tc_final(s2, y2, h1, dinv, b2.reshape(1, hdim),
                     ln_gamma.reshape(1, hdim), ln_beta.reshape(1, hdim),
                     hidden_state[0], W_ih, W_hh,
                     b_ih.reshape(1, -1), b_hh.reshape(1, -1))
    return hnew, hnew[None, :, :]
